# baseline (speedup 1.0000x reference)
import functools

import jax
import jax.numpy as jnp
from jax import lax
from jax.experimental import pallas as pl
from jax.experimental.pallas import tpu as pltpu
from jax.experimental.pallas import tpu_sc as plsc

NN = 50000
HH = 64
HF = HH // 2
GG = 512
LL = 4
BN = 512
NP = 50176
NBG = NP // BN
EE = 800000
NCORE = 2
NSUB = 16
GRP = 384
SUB = 128
NSS = GRP // SUB
NITER = 66
ES = NITER * 2 * GRP
EP = NSUB * ES
TPR = NP // NSUB
F32 = jnp.float32


def _edge_segsum_body(r2, comb, agg2, idx, rows, acc, gsem, ssem, isem):
    c = lax.axis_index("c")
    s = lax.axis_index("s")
    dummy = r2.at[0].at[pl.ds(0, GRP)]

    z16 = jnp.zeros((16,), F32)

    def memset_row(i, carry):
        rows[i, 0:16] = z16
        rows[i, 16:32] = z16
        return carry

    lax.fori_loop(0, 2 * GRP, memset_row, 0)
    zblk = 2 * GRP
    for k in range(TPR // zblk):
        pltpu.sync_copy(rows.at[pl.ds(0, zblk)],
                        acc.at[pl.ds(s * TPR + k * zblk, zblk)])
    rem = TPR - (TPR // zblk) * zblk
    if rem:
        pltpu.sync_copy(rows.at[pl.ds(0, rem)],
                        acc.at[pl.ds(s * TPR + (TPR // zblk) * zblk, rem)])
    plsc.subcore_barrier()

    rh = r2.at[c]
    bufa = rows.at[pl.ds(0, GRP)]
    bufb = rows.at[pl.ds(GRP, GRP)]

    def fire_gathers(ibase, buf):
        for j in range(NSS):
            pltpu.async_copy(
                rh.at[idx.at[ibase + j]], buf.at[pl.ds(j * SUB, SUB)], gsem
            )

    def fire_scatters(ibase, buf):
        for j in range(NSS):
            pltpu.async_copy(
                buf.at[pl.ds(j * SUB, SUB)], acc.at[idx.at[ibase + j]],
                ssem, add=True,
            )

    def drain_gathers():
        pltpu.make_async_copy(dummy, bufa, gsem).wait()

    def drain_scatters():
        pltpu.make_async_copy(dummy, acc.at[pl.ds(0, GRP)], ssem).wait()

    pltpu.sync_copy(comb.at[s * NITER], idx.at[pl.ds(0, 4 * NSS)])
    fire_gathers(0, bufa)

    def body(m, carry):
        slot = lax.rem(m, 2) * (4 * NSS)
        nslot = lax.rem(m + 1, 2) * (4 * NSS)
        drain_gathers()

        @pl.when(m >= 1)
        def _():
            drain_scatters()

        @pl.when(m + 1 < NITER)
        def _():
            pltpu.async_copy(
                comb.at[s * NITER + m + 1], idx.at[pl.ds(nslot, 4 * NSS)], isem
            )
        fire_gathers(slot + 2 * NSS, bufb)
        fire_scatters(slot + NSS, bufa)
        drain_gathers()
        drain_scatters()

        @pl.when(m + 1 < NITER)
        def _():
            pltpu.make_async_copy(
                comb.at[0], idx.at[pl.ds(nslot, 4 * NSS)], isem
            ).wait()
            fire_gathers(nslot, bufa)
        fire_scatters(slot + 3 * NSS, bufb)
        return carry

    lax.fori_loop(0, NITER, body, 0)
    drain_scatters()
    plsc.subcore_barrier()
    pltpu.sync_copy(
        acc.at[pl.ds(s * TPR, TPR)], agg2.at[c].at[pl.ds(s * TPR, TPR)]
    )


_SC_MESH = plsc.VectorSubcoreMesh(
    core_axis_name="c", subcore_axis_name="s", num_cores=NCORE, num_subcores=NSUB
)

_edge_segsum = pl.kernel(
    _edge_segsum_body,
    out_type=jax.ShapeDtypeStruct((NCORE, NP, HF), F32),
    mesh=_SC_MESH,
    scratch_types=[
        pltpu.VMEM((8 * NSS, SUB), jnp.int32),
        pltpu.VMEM((2 * GRP, HF), F32),
        pltpu.VMEM_SHARED((NP, HF), F32),
        pltpu.SemaphoreType.DMA,
        pltpu.SemaphoreType.DMA,
        pltpu.SemaphoreType.DMA,
    ],
    compiler_params=pltpu.CompilerParams(use_tc_tiling_on_sc=False),
)


def _pre_body(h_ref, b_ref, vn_ref, hv_ref, r2_ref):
    i = pl.program_id(0)
    bids = b_ref[0, 0, :].reshape(1, BN)
    gi = lax.broadcasted_iota(jnp.int32, (GG, BN), 0)
    oht = (gi == bids).astype(F32)
    vnb = lax.dot_general(
        oht, vn_ref[...], (((0,), (0,)), ((), ())), preferred_element_type=F32
    )
    hv = h_ref[...] + vnb
    hv_ref[...] = hv
    rid = i * BN + lax.broadcasted_iota(jnp.int32, (BN, 1), 0)
    valid = (rid < NN).astype(F32)
    r = jnp.maximum(hv, 0.0) * valid
    r2_ref[0] = r[:, :HF]
    r2_ref[1] = r[:, HF:]


_pre = pl.pallas_call(
    _pre_body,
    grid=(NBG,),
    in_specs=[
        pl.BlockSpec((BN, HH), lambda i: (i, 0)),
        pl.BlockSpec((1, 1, BN), lambda i: (i, 0, 0)),
        pl.BlockSpec((GG, HH), lambda i: (0, 0)),
    ],
    out_specs=[
        pl.BlockSpec((BN, HH), lambda i: (i, 0)),
        pl.BlockSpec((NCORE, BN, HF), lambda i: (0, i, 0)),
    ],
    out_shape=[
        jax.ShapeDtypeStruct((NP, HH), F32),
        jax.ShapeDtypeStruct((NCORE, NP, HF), F32),
    ],
)


def _post_body(hv_ref, agg_ref, h_ref, b_ref, w1_ref, b1_ref, w2_ref, b2_ref,
               sc_ref, sh_ref, eps_ref, z_ref, pool_ref, vt_ref, *, final):
    i = pl.program_id(0)
    agg = jnp.concatenate([agg_ref[k] for k in range(NCORE)], axis=1)
    zin = (1.0 + eps_ref[0, 0]) * hv_ref[...] + agg
    t = jnp.maximum(
        jnp.dot(zin, w1_ref[...], preferred_element_type=F32) + b1_ref[...], 0.0
    )
    z = jnp.dot(t, w2_ref[...], preferred_element_type=F32) + b2_ref[...]
    z = z * sc_ref[...] + sh_ref[...]
    if not final:
        z = jnp.maximum(z, 0.0)
    z_ref[...] = z
    bids = b_ref[0, 0, :].reshape(1, BN)
    gi = lax.broadcasted_iota(jnp.int32, (GG, BN), 0)
    oht = (gi == bids).astype(F32)
    pool_blk = jnp.dot(oht, z, preferred_element_type=F32)
    vt_blk = jnp.dot(oht, h_ref[...], preferred_element_type=F32)

    @pl.when(i == 0)
    def _():
        pool_ref[...] = jnp.zeros_like(pool_ref)
        vt_ref[...] = jnp.zeros_like(vt_ref)

    pool_ref[...] += pool_blk
    vt_ref[...] += vt_blk


def _make_post(final):
    return pl.pallas_call(
        functools.partial(_post_body, final=final),
        grid=(NBG,),
        in_specs=[
            pl.BlockSpec((BN, HH), lambda i: (i, 0)),
            pl.BlockSpec((NCORE, BN, HF), lambda i: (0, i, 0)),
            pl.BlockSpec((BN, HH), lambda i: (i, 0)),
            pl.BlockSpec((1, 1, BN), lambda i: (i, 0, 0)),
            pl.BlockSpec((HH, 2 * HH), lambda i: (0, 0)),
            pl.BlockSpec((1, 2 * HH), lambda i: (0, 0)),
            pl.BlockSpec((2 * HH, HH), lambda i: (0, 0)),
            pl.BlockSpec((1, HH), lambda i: (0, 0)),
            pl.BlockSpec((1, HH), lambda i: (0, 0)),
            pl.BlockSpec((1, HH), lambda i: (0, 0)),
            pl.BlockSpec((1, 1), lambda i: (0, 0)),
        ],
        out_specs=[
            pl.BlockSpec((BN, HH), lambda i: (i, 0)),
            pl.BlockSpec((GG, HH), lambda i: (0, 0)),
            pl.BlockSpec((GG, HH), lambda i: (0, 0)),
        ],
        out_shape=[
            jax.ShapeDtypeStruct((NP, HH), F32),
            jax.ShapeDtypeStruct((GG, HH), F32),
            jax.ShapeDtypeStruct((GG, HH), F32),
        ],
    )


_post_mid = _make_post(final=False)
_post_final = _make_post(final=True)


def _vn_body(vt_ref, vn_ref, wv1_ref, bv1_ref, wv2_ref, bv2_ref, out_ref):
    vt = vt_ref[...] + vn_ref[...]
    t = jnp.maximum(
        jnp.dot(vt, wv1_ref[...], preferred_element_type=F32) + bv1_ref[...], 0.0
    )
    o = jnp.dot(t, wv2_ref[...], preferred_element_type=F32) + bv2_ref[...]
    out_ref[...] = jnp.maximum(o, 0.0)


_vn_update = pl.pallas_call(
    _vn_body,
    out_shape=jax.ShapeDtypeStruct((GG, HH), F32),
)


def kernel(x, edge_index, batch, W1, b1, W2, b2, eps, bn_scale, bn_shift,
           Wv1, bv1, Wv2, bv2, vn_emb):
    h = jnp.pad(x.astype(F32), ((0, NP - NN), (0, 0)))
    b3 = jnp.pad(batch.astype(jnp.int32), (0, NP - NN),
                 constant_values=GG).reshape(NBG, 1, BN)
    srcp = jnp.pad(edge_index[0].astype(jnp.int32), (0, EP - EE),
                   constant_values=NP - 1).reshape(NSUB, NITER, 2, NSS, SUB)
    dstp = jnp.pad(edge_index[1].astype(jnp.int32), (0, EP - EE),
                   constant_values=NP - 1).reshape(NSUB, NITER, 2, NSS, SUB)
    comb = jnp.concatenate(
        [srcp[:, :, 0], dstp[:, :, 0], srcp[:, :, 1], dstp[:, :, 1]], axis=2
    ).reshape(NSUB * NITER, 4 * NSS, SUB)
    vn = jnp.broadcast_to(vn_emb.astype(F32), (GG, HH))

    pooled = []
    for l in range(LL):
        hv, r2 = _pre(h, b3, vn)
        agg2 = _edge_segsum(r2, comb)
        post = _post_final if l == LL - 1 else _post_mid
        z, pool_l, vt_sum = post(
            hv, agg2, h, b3,
            W1[l], b1[l].reshape(1, 2 * HH), W2[l], b2[l].reshape(1, HH),
            bn_scale[l].reshape(1, HH), bn_shift[l].reshape(1, HH),
            eps[l].reshape(1, 1),
        )
        pooled.append(pool_l)
        if l < LL - 1:
            vn = _vn_update(
                vt_sum, vn,
                Wv1[l], bv1[l].reshape(1, 2 * HH),
                Wv2[l], bv2[l].reshape(1, HH),
            )
        h = z
    return jnp.concatenate(pooled, axis=1)


# speedup vs baseline: 1.2033x; 1.2033x over previous
import functools

import jax
import jax.numpy as jnp
from jax import lax
from jax.experimental import pallas as pl
from jax.experimental.pallas import tpu as pltpu
from jax.experimental.pallas import tpu_sc as plsc

NN = 50000
HH = 64
HF = HH // 2
GG = 512
LL = 4
BN = 512
NP = 50176
NBG = NP // BN
EE = 800000
NCORE = 2
NSUB = 16
GRP = 256
SUB = 128
NSS = GRP // SUB
NITER = 98
ES = NITER * 2 * GRP
EP = NSUB * ES
TPR = NP // NSUB
F32 = jnp.float32


def _edge_segsum_body(r2, comb, agg2, idx, rows, acc, gsem, ssem, isem):
    c = lax.axis_index("c")
    s = lax.axis_index("s")
    dummy = r2.at[0].at[pl.ds(0, GRP)]

    z16 = jnp.zeros((16,), F32)

    def memset_row(i, carry):
        rows[i, 0:16] = z16
        rows[i, 16:32] = z16
        return carry

    lax.fori_loop(0, 2 * GRP, memset_row, 0)
    zblk = 2 * GRP
    for k in range(TPR // zblk):
        pltpu.sync_copy(rows.at[pl.ds(0, zblk)],
                        acc.at[pl.ds(s * TPR + k * zblk, zblk)])
    rem = TPR - (TPR // zblk) * zblk
    if rem:
        pltpu.sync_copy(rows.at[pl.ds(0, rem)],
                        acc.at[pl.ds(s * TPR + (TPR // zblk) * zblk, rem)])
    plsc.subcore_barrier()

    rh = r2.at[c]
    bufa = rows.at[pl.ds(0, GRP)]
    bufb = rows.at[pl.ds(GRP, GRP)]

    def fire_gathers(ibase, buf):
        for j in range(NSS):
            pltpu.async_copy(
                rh.at[idx.at[ibase + j]], buf.at[pl.ds(j * SUB, SUB)], gsem
            )

    def fire_scatters(ibase, buf):
        for j in range(NSS):
            pltpu.async_copy(
                buf.at[pl.ds(j * SUB, SUB)], acc.at[idx.at[ibase + j]],
                ssem, add=True,
            )

    def drain_gathers():
        pltpu.make_async_copy(dummy, bufa, gsem).wait()

    def drain_scatters():
        pltpu.make_async_copy(dummy, acc.at[pl.ds(0, GRP)], ssem).wait()

    pltpu.sync_copy(comb.at[s * NITER], idx.at[pl.ds(0, 4 * NSS)])
    fire_gathers(0, bufa)

    def body(m, carry):
        slot = lax.rem(m, 2) * (4 * NSS)
        nslot = lax.rem(m + 1, 2) * (4 * NSS)
        drain_gathers()

        @pl.when(m >= 1)
        def _():
            drain_scatters()

        @pl.when(m + 1 < NITER)
        def _():
            pltpu.async_copy(
                comb.at[s * NITER + m + 1], idx.at[pl.ds(nslot, 4 * NSS)], isem
            )
        fire_gathers(slot + 2 * NSS, bufb)
        fire_scatters(slot + NSS, bufa)
        drain_gathers()
        drain_scatters()

        @pl.when(m + 1 < NITER)
        def _():
            pltpu.make_async_copy(
                comb.at[0], idx.at[pl.ds(nslot, 4 * NSS)], isem
            ).wait()
            fire_gathers(nslot, bufa)
        fire_scatters(slot + 3 * NSS, bufb)
        return carry

    lax.fori_loop(0, NITER, body, 0)
    drain_scatters()
    plsc.subcore_barrier()
    pltpu.sync_copy(
        acc.at[pl.ds(s * TPR, TPR)], agg2.at[c].at[pl.ds(s * TPR, TPR)]
    )


_SC_MESH = plsc.VectorSubcoreMesh(
    core_axis_name="c", subcore_axis_name="s", num_cores=NCORE, num_subcores=NSUB
)

_edge_segsum = pl.kernel(
    _edge_segsum_body,
    out_type=jax.ShapeDtypeStruct((NCORE, NP, HF), F32),
    mesh=_SC_MESH,
    scratch_types=[
        pltpu.VMEM((8 * NSS, SUB), jnp.int32),
        pltpu.VMEM((2 * GRP, HF), F32),
        pltpu.VMEM_SHARED((NP, HF), F32),
        pltpu.SemaphoreType.DMA,
        pltpu.SemaphoreType.DMA,
        pltpu.SemaphoreType.DMA,
    ],
    compiler_params=pltpu.CompilerParams(use_tc_tiling_on_sc=False),
)


def _pre_body(h_ref, b_ref, vn_ref, hv_ref, r2_ref):
    i = pl.program_id(0)
    bids = b_ref[0, 0, :].reshape(1, BN)
    gi = lax.broadcasted_iota(jnp.int32, (GG, BN), 0)
    oht = (gi == bids).astype(F32)
    vnb = lax.dot_general(
        oht, vn_ref[...], (((0,), (0,)), ((), ())), preferred_element_type=F32
    )
    hv = h_ref[...] + vnb
    hv_ref[...] = hv
    rid = i * BN + lax.broadcasted_iota(jnp.int32, (BN, 1), 0)
    valid = (rid < NN).astype(F32)
    r = jnp.maximum(hv, 0.0) * valid
    r2_ref[0] = r[:, :HF]
    r2_ref[1] = r[:, HF:]


_pre = pl.pallas_call(
    _pre_body,
    grid=(NBG,),
    in_specs=[
        pl.BlockSpec((BN, HH), lambda i: (i, 0)),
        pl.BlockSpec((1, 1, BN), lambda i: (i, 0, 0)),
        pl.BlockSpec((GG, HH), lambda i: (0, 0)),
    ],
    out_specs=[
        pl.BlockSpec((BN, HH), lambda i: (i, 0)),
        pl.BlockSpec((NCORE, BN, HF), lambda i: (0, i, 0)),
    ],
    out_shape=[
        jax.ShapeDtypeStruct((NP, HH), F32),
        jax.ShapeDtypeStruct((NCORE, NP, HF), F32),
    ],
)


def _post_body(hv_ref, agg_ref, h_ref, b_ref, w1_ref, b1_ref, w2_ref, b2_ref,
               sc_ref, sh_ref, eps_ref, z_ref, pool_ref, vt_ref, *, final):
    i = pl.program_id(0)
    agg = jnp.concatenate([agg_ref[k] for k in range(NCORE)], axis=1)
    zin = (1.0 + eps_ref[0, 0]) * hv_ref[...] + agg
    t = jnp.maximum(
        jnp.dot(zin, w1_ref[...], preferred_element_type=F32) + b1_ref[...], 0.0
    )
    z = jnp.dot(t, w2_ref[...], preferred_element_type=F32) + b2_ref[...]
    z = z * sc_ref[...] + sh_ref[...]
    if not final:
        z = jnp.maximum(z, 0.0)
    z_ref[...] = z
    bids = b_ref[0, 0, :].reshape(1, BN)
    gi = lax.broadcasted_iota(jnp.int32, (GG, BN), 0)
    oht = (gi == bids).astype(F32)
    pool_blk = jnp.dot(oht, z, preferred_element_type=F32)
    vt_blk = jnp.dot(oht, h_ref[...], preferred_element_type=F32)

    @pl.when(i == 0)
    def _():
        pool_ref[...] = jnp.zeros_like(pool_ref)
        vt_ref[...] = jnp.zeros_like(vt_ref)

    pool_ref[...] += pool_blk
    vt_ref[...] += vt_blk


def _make_post(final):
    return pl.pallas_call(
        functools.partial(_post_body, final=final),
        grid=(NBG,),
        in_specs=[
            pl.BlockSpec((BN, HH), lambda i: (i, 0)),
            pl.BlockSpec((NCORE, BN, HF), lambda i: (0, i, 0)),
            pl.BlockSpec((BN, HH), lambda i: (i, 0)),
            pl.BlockSpec((1, 1, BN), lambda i: (i, 0, 0)),
            pl.BlockSpec((HH, 2 * HH), lambda i: (0, 0)),
            pl.BlockSpec((1, 2 * HH), lambda i: (0, 0)),
            pl.BlockSpec((2 * HH, HH), lambda i: (0, 0)),
            pl.BlockSpec((1, HH), lambda i: (0, 0)),
            pl.BlockSpec((1, HH), lambda i: (0, 0)),
            pl.BlockSpec((1, HH), lambda i: (0, 0)),
            pl.BlockSpec((1, 1), lambda i: (0, 0)),
        ],
        out_specs=[
            pl.BlockSpec((BN, HH), lambda i: (i, 0)),
            pl.BlockSpec((GG, HH), lambda i: (0, 0)),
            pl.BlockSpec((GG, HH), lambda i: (0, 0)),
        ],
        out_shape=[
            jax.ShapeDtypeStruct((NP, HH), F32),
            jax.ShapeDtypeStruct((GG, HH), F32),
            jax.ShapeDtypeStruct((GG, HH), F32),
        ],
    )


_post_mid = _make_post(final=False)
_post_final = _make_post(final=True)


def _vn_body(vt_ref, vn_ref, wv1_ref, bv1_ref, wv2_ref, bv2_ref, out_ref):
    vt = vt_ref[...] + vn_ref[...]
    t = jnp.maximum(
        jnp.dot(vt, wv1_ref[...], preferred_element_type=F32) + bv1_ref[...], 0.0
    )
    o = jnp.dot(t, wv2_ref[...], preferred_element_type=F32) + bv2_ref[...]
    out_ref[...] = jnp.maximum(o, 0.0)


_vn_update = pl.pallas_call(
    _vn_body,
    out_shape=jax.ShapeDtypeStruct((GG, HH), F32),
)


def kernel(x, edge_index, batch, W1, b1, W2, b2, eps, bn_scale, bn_shift,
           Wv1, bv1, Wv2, bv2, vn_emb):
    h = jnp.pad(x.astype(F32), ((0, NP - NN), (0, 0)))
    b3 = jnp.pad(batch.astype(jnp.int32), (0, NP - NN),
                 constant_values=GG).reshape(NBG, 1, BN)
    srcp = jnp.pad(edge_index[0].astype(jnp.int32), (0, EP - EE),
                   constant_values=NP - 1).reshape(NSUB, NITER, 2, NSS, SUB)
    dstp = jnp.pad(edge_index[1].astype(jnp.int32), (0, EP - EE),
                   constant_values=NP - 1).reshape(NSUB, NITER, 2, NSS, SUB)
    comb = jnp.concatenate(
        [srcp[:, :, 0], dstp[:, :, 0], srcp[:, :, 1], dstp[:, :, 1]], axis=2
    ).reshape(NSUB * NITER, 4 * NSS, SUB)
    vn = jnp.broadcast_to(vn_emb.astype(F32), (GG, HH))

    pooled = []
    for l in range(LL):
        hv, r2 = _pre(h, b3, vn)
        agg2 = _edge_segsum(r2, comb)
        post = _post_final if l == LL - 1 else _post_mid
        z, pool_l, vt_sum = post(
            hv, agg2, h, b3,
            W1[l], b1[l].reshape(1, 2 * HH), W2[l], b2[l].reshape(1, HH),
            bn_scale[l].reshape(1, HH), bn_shift[l].reshape(1, HH),
            eps[l].reshape(1, 1),
        )
        pooled.append(pool_l)
        if l < LL - 1:
            vn = _vn_update(
                vt_sum, vn,
                Wv1[l], bv1[l].reshape(1, 2 * HH),
                Wv2[l], bv2[l].reshape(1, HH),
            )
        h = z
    return jnp.concatenate(pooled, axis=1)
